# baseline (speedup 1.0000x reference)
import jax
import jax.numpy as jnp
from jax.experimental import pallas as pl

VX, VY, VZ = 0.2, 0.2, 4.0
PCR = (0.0, -40.0, -3.0, 70.4, 40.0, 1.0)
CZ, CY, CX = 1, 400, 352
C1, C2 = 64, 128
X_OFF = VX / 2 + PCR[0]
Y_OFF = VY / 2 + PCR[1]
Z_OFF = VZ / 2 + PCR[2]


def _bn_relu(h, g, b):
    m = jnp.mean(h, axis=0)
    v = jnp.var(h, axis=0)
    return jax.nn.relu((h - m) / jnp.sqrt(v + 1e-3) * g + b)


def _mask_kernel(v2_ref, cnt_ref, o_ref):
    o_ref[...] = jnp.where(cnt_ref[...] > 0, v2_ref[...], 0.0)


def kernel(points, W1, g1, b1, W2, g2, b2, batch_size):
    B = 2
    bidx = points[:, 0].astype(jnp.int32)
    xyz = points[:, 1:4]
    feats = points[:, 1:5]
    cx = jnp.clip(jnp.floor((points[:, 1] - PCR[0]) / VX).astype(jnp.int32), 0, CX - 1)
    cy = jnp.clip(jnp.floor((points[:, 2] - PCR[1]) / VY).astype(jnp.int32), 0, CY - 1)
    cz = jnp.clip(jnp.floor((points[:, 3] - PCR[2]) / VZ).astype(jnp.int32), 0, CZ - 1)
    flat = ((bidx * CZ + cz) * CY + cy) * CX + cx
    L = B * CZ * CY * CX
    ones = jnp.ones((points.shape[0],), dtype=jnp.float32)
    cnt = jax.ops.segment_sum(ones, flat, num_segments=L)
    sums = jax.ops.segment_sum(xyz, flat, num_segments=L)
    vmean = sums / jnp.maximum(cnt, 1.0)[:, None]
    points_mean = vmean[flat]
    f_cluster = xyz - points_mean
    f_center = jnp.stack([
        points[:, 1] - (cx.astype(jnp.float32) * VX + X_OFF),
        points[:, 2] - (cy.astype(jnp.float32) * VY + Y_OFF),
        points[:, 3] - (cz.astype(jnp.float32) * VZ + Z_OFF),
    ], axis=1)
    features = jnp.concatenate([feats, f_cluster, f_center], axis=1)
    p1 = _bn_relu(features @ W1, g1, b1)
    v1 = jax.ops.segment_max(p1, flat, num_segments=L)
    v1 = jnp.where(cnt[:, None] > 0, v1, 0.0)
    features2 = jnp.concatenate([p1, v1[flat]], axis=1)
    p2 = _bn_relu(features2 @ W2, g2, b2)
    v2 = jax.ops.segment_max(p2, flat, num_segments=L)

    rows = 512
    out = pl.pallas_call(
        _mask_kernel,
        grid=(L // rows,),
        in_specs=[
            pl.BlockSpec((rows, C2), lambda i: (i, 0)),
            pl.BlockSpec((rows, 1), lambda i: (i, 0)),
        ],
        out_specs=pl.BlockSpec((rows, C2), lambda i: (i, 0)),
        out_shape=jax.ShapeDtypeStruct((L, C2), jnp.float32),
    )(v2, cnt[:, None])
    return out


# speedup vs baseline: 1.0673x; 1.0673x over previous
import functools
import jax
import jax.numpy as jnp
from jax import lax
from jax.experimental import pallas as pl
from jax.experimental.pallas import tpu as pltpu, tpu_sc as plsc

VX, VY, VZ = 0.2, 0.2, 4.0
PCR = (0.0, -40.0, -3.0, 70.4, 40.0, 1.0)
CZ, CY, CX = 1, 400, 352
B = 2
N = 262144
C1, C2 = 64, 128
X_OFF = VX / 2 + PCR[0]
Y_OFF = VY / 2 + PCR[1]
Z_OFF = VZ / 2 + PCR[2]
L = B * CZ * CY * CX
HL = L // 2
NW = 32
NHALF = N // 2
WPTS = N // NW
CH = 4096
VR = HL // 16
F = 2048
CAP = 133120
SUB1, NS1, CV1, DUMP1 = 1100, 8, 1152, 1100
SUB2, NS2, CV2, DUMP2 = 550, 16, 576, 550
GCH = 128
BN = 2048
NB = N // BN

_mesh = plsc.VectorSubcoreMesh(core_axis_name="c", subcore_axis_name="s")
_scp = pltpu.CompilerParams(needs_layout_passes=False)


def _prep_body(p_ref, flat_ref, pt4_ref):
    x = p_ref[:, 1]
    y = p_ref[:, 2]
    z = p_ref[:, 3]
    b = p_ref[:, 0].astype(jnp.int32)
    cx = jnp.clip(jnp.floor((x - PCR[0]) / VX).astype(jnp.int32), 0, CX - 1)
    cy = jnp.clip(jnp.floor((y - PCR[1]) / VY).astype(jnp.int32), 0, CY - 1)
    cz = jnp.clip(jnp.floor((z - PCR[2]) / VZ).astype(jnp.int32), 0, CZ - 1)
    flat = ((b * CZ + cz) * CY + cy) * CX + cx
    flat_ref[...] = flat[:, None]
    pt4_ref[...] = jnp.stack([x, y, z, jnp.ones_like(x)], axis=1)


def _x1(p_ref, fc_ref):
    x = p_ref[:, 1]
    y = p_ref[:, 2]
    z = p_ref[:, 3]
    cx = jnp.clip(jnp.floor((x - PCR[0]) / VX).astype(jnp.int32), 0, CX - 1)
    cy = jnp.clip(jnp.floor((y - PCR[1]) / VY).astype(jnp.int32), 0, CY - 1)
    cz = jnp.clip(jnp.floor((z - PCR[2]) / VZ).astype(jnp.int32), 0, CZ - 1)
    fcen = jnp.stack([
        x - (cx.astype(jnp.float32) * VX + X_OFF),
        y - (cy.astype(jnp.float32) * VY + Y_OFF),
        z - (cz.astype(jnp.float32) * VZ + Z_OFF),
    ], axis=1)
    return jnp.concatenate([p_ref[:, 1:5], fc_ref[:, 0:3], fcen], axis=1)


def _gram1_body(p_ref, fc_ref, g_ref, acc):
    i = pl.program_id(0)

    @pl.when(i == 0)
    def _():
        acc[...] = jnp.zeros_like(acc)

    X = _x1(p_ref, fc_ref)
    ones = jnp.ones((X.shape[0], 1), jnp.float32)
    zeros = jnp.zeros((X.shape[0], 5), jnp.float32)
    Xe = jnp.concatenate([X, ones, zeros], axis=1)
    acc[...] += lax.dot_general(Xe, Xe, (((0,), (0,)), ((), ())),
                                preferred_element_type=jnp.float32)

    @pl.when(i == NB - 1)
    def _():
        g_ref[...] = acc[...]


def _apply1_body(p_ref, fc_ref, w_ref, sc_ref, sh_ref, o_ref):
    X = _x1(p_ref, fc_ref)
    h = jnp.dot(X, w_ref[...], preferred_element_type=jnp.float32)
    p1 = jax.nn.relu(h * sc_ref[...] + sh_ref[...])
    o_ref[...] = jnp.concatenate(
        [p1, jnp.zeros((p1.shape[0], C2 - C1), jnp.float32)], axis=1)


def _gram2_body(x2_ref, g_ref, acc):
    i = pl.program_id(0)

    @pl.when(i == 0)
    def _():
        acc[...] = jnp.zeros_like(acc)

    X = x2_ref[...]
    ones = jnp.ones((X.shape[0], 1), jnp.float32)
    zeros = jnp.zeros((X.shape[0], 7), jnp.float32)
    Xe = jnp.concatenate([X, ones, zeros], axis=1)
    acc[...] += lax.dot_general(Xe, Xe, (((0,), (0,)), ((), ())),
                                preferred_element_type=jnp.float32)

    @pl.when(i == NB - 1)
    def _():
        g_ref[...] = acc[...]


def _apply2_body(x2_ref, w_ref, sc_ref, sh_ref, o_ref):
    h = jnp.dot(x2_ref[...], w_ref[...], preferred_element_type=jnp.float32)
    o_ref[...] = jax.nn.relu(h * sc_ref[...] + sh_ref[...])


@functools.partial(
    pl.kernel,
    out_type=(
        jax.ShapeDtypeStruct((N * 4,), jnp.float32),
        jax.ShapeDtypeStruct((L * 4,), jnp.float32),
        jax.ShapeDtypeStruct((NW * CAP,), jnp.int32),
        jax.ShapeDtypeStruct((NW * 16,), jnp.int32),
    ),
    mesh=_mesh,
    compiler_params=_scp,
    scratch_types=[
        pltpu.VMEM((CH,), jnp.int32),
        pltpu.VMEM((CH * 4,), jnp.float32),
        pltpu.VMEM((CH * 4,), jnp.int32),
        pltpu.VMEM((CH * 4,), jnp.float32),
        pltpu.VMEM((CH * 4,), jnp.float32),
        pltpu.VMEM((F + 16,), jnp.int32),
        pltpu.VMEM((16,), jnp.int32),
        pltpu.VMEM((VR // 2,), jnp.float32),
        pltpu.VMEM_SHARED((HL * 4,), jnp.float32),
        pltpu.SemaphoreType.DMA,
    ],
)
def _sc_a(flat_h, pt4_h, fclu_h, vcan_h, lists_h, counts_h,
          fiv, valw, eix, rows, ob, lbuf, cb, zb, canvas, sem):
    cid = lax.axis_index("c")
    sid = lax.axis_index("s")
    w = cid * 16 + sid
    lanes = lax.iota(jnp.int32, 16)

    def _zb(i, _):
        zb[pl.ds(i * 16, 16)] = jnp.zeros((16,), jnp.float32)
        return 0
    lax.fori_loop(0, (VR // 2) // 16, _zb, 0)

    def _zc(i, _):
        pltpu.sync_copy(zb, canvas.at[pl.ds(sid * VR * 4 + i * (VR // 2), VR // 2)])
        return 0
    lax.fori_loop(0, 8, _zc, 0)
    plsc.subcore_barrier()

    for t in range(WPTS // CH):
        base = w * WPTS + t * CH
        pltpu.sync_copy(flat_h.at[pl.ds(base, CH)], fiv)
        pltpu.sync_copy(pt4_h.at[pl.ds(base * 4, CH * 4)], valw)

        def _bx(i, _):
            f = fiv[pl.ds(i * 16, 16)]
            e = (f - cid * HL) * 4
            for c in range(4):
                plsc.store_scatter(eix, [lanes * 4 + (i * 64 + c)], e + c)
            return 0
        lax.fori_loop(0, CH // 16, _bx, 0)
        pltpu.sync_copy(valw, canvas.at[eix], add=True)
    plsc.subcore_barrier()

    def _rc(i, _):
        pltpu.sync_copy(canvas.at[pl.ds(sid * VR * 4 + i * (VR // 2), VR // 2)], zb)
        pltpu.sync_copy(zb, vcan_h.at[pl.ds(cid * HL * 4 + sid * VR * 4 + i * (VR // 2), VR // 2)])
        return 0
    lax.fori_loop(0, 8, _rc, 0)

    for t in range(WPTS // CH):
        base = w * WPTS + t * CH
        pltpu.sync_copy(flat_h.at[pl.ds(base, CH)], fiv)
        pltpu.sync_copy(pt4_h.at[pl.ds(base * 4, CH * 4)], valw)

        def _bg(i, _):
            f = fiv[pl.ds(i * 16, 16)]
            e = f * 4
            for c in range(4):
                plsc.store_scatter(eix, [lanes * 4 + (i * 64 + c)], e + c)
            return 0
        lax.fori_loop(0, CH // 16, _bg, 0)
        pltpu.async_copy(vcan_h.at[eix], rows, sem).wait()

        def _fc(i, _):
            i64 = i * 64
            sx = plsc.load_gather(rows, [i64 + lanes * 4 + 0])
            sy = plsc.load_gather(rows, [i64 + lanes * 4 + 1])
            sz = plsc.load_gather(rows, [i64 + lanes * 4 + 2])
            cv = plsc.load_gather(rows, [i64 + lanes * 4 + 3])
            x = plsc.load_gather(valw, [i64 + lanes * 4 + 0])
            y = plsc.load_gather(valw, [i64 + lanes * 4 + 1])
            z = plsc.load_gather(valw, [i64 + lanes * 4 + 2])
            inv = 1.0 / jnp.maximum(cv, 1.0)
            plsc.store_scatter(ob, [i64 + lanes * 4 + 0], x - sx * inv)
            plsc.store_scatter(ob, [i64 + lanes * 4 + 1], y - sy * inv)
            plsc.store_scatter(ob, [i64 + lanes * 4 + 2], z - sz * inv)
            plsc.store_scatter(ob, [i64 + lanes * 4 + 3], jnp.zeros((16,), jnp.float32))
            return 0
        lax.fori_loop(0, CH // 16, _fc, 0)
        pltpu.sync_copy(ob, fclu_h.at[pl.ds(base * 4, CH * 4)])

    dump_pat = jnp.full((16,), VR, jnp.int32) << 18

    def _lb(i, _):
        lbuf[pl.ds(i * 16, 16)] = dump_pat
        return 0
    lax.fori_loop(0, (F + 16) // 16, _lb, 0)

    bb = cid * NHALF

    def _chunk(t, carry):
        pltpu.sync_copy(flat_h.at[pl.ds(bb + t * CH, CH)], fiv)

        def _v(i, cr):
            cnt, fl = cr
            f = fiv[pl.ds(i * 16, 16)]
            lid = f - w * VR
            m = jnp.logical_and(lid >= 0, lid < VR)
            pos = bb + t * CH + i * 16 + lanes
            packed = jnp.bitwise_or(pos, lid << 18)
            offs = plsc.cumsum(jnp.where(m, 1, 0))
            slots = cnt + offs - 1
            plsc.store_scatter(lbuf, [slots], packed, mask=m)
            cnt = cnt + jnp.max(offs)

            def _flush(c):
                cc, ff = c
                ffa = pl.multiple_of(ff, F)
                pltpu.sync_copy(lbuf.at[pl.ds(0, F)], lists_h.at[pl.ds(w * CAP + ffa, F)])
                tail = lbuf[pl.ds(F, 16)]
                lbuf[pl.ds(0, 16)] = tail
                return (cc - F, ff + F)

            return lax.cond(cnt >= F, _flush, lambda c: c, (cnt, fl))
        return lax.fori_loop(0, CH // 16, _v, carry)

    cnt, fl = lax.fori_loop(0, NHALF // CH, _chunk, (0, 0))
    fla = pl.multiple_of(fl, F)
    pltpu.sync_copy(lbuf.at[pl.ds(0, F)], lists_h.at[pl.ds(w * CAP + fla, F)])
    padded = fl + ((cnt + 255) // 256) * 256
    cb[pl.ds(0, 16)] = jnp.broadcast_to(padded, (16,)).astype(jnp.int32)
    pltpu.sync_copy(cb, counts_h.at[pl.ds(w * 16, 16)])


@functools.partial(
    pl.kernel,
    out_type=jax.ShapeDtypeStruct((N + 8, C2), jnp.float32),
    mesh=_mesh,
    compiler_params=_scp,
    scratch_types=[
        pltpu.VMEM((256,), jnp.int32),
        pltpu.VMEM((GCH + 16,), jnp.int32),
        pltpu.VMEM((GCH + 16,), jnp.int32),
        pltpu.VMEM((GCH,), jnp.int32),
        pltpu.VMEM((GCH,), jnp.int32),
        pltpu.VMEM((GCH, C2), jnp.float32),
        pltpu.VMEM((GCH, C2), jnp.float32),
        pltpu.VMEM((CV1 * C1,), jnp.float32),
        pltpu.VMEM((16,), jnp.int32),
        pltpu.SemaphoreType.DMA,
    ],
)
def _sc_b(p1_h, lists_h, counts_h, x2_h,
          lch, gpx, glx, gpg, si, rb, vb2, canvas, cb, sem):
    cid = lax.axis_index("c")
    sid = lax.axis_index("s")
    w = cid * 16 + sid
    lanes = lax.iota(jnp.int32, 16)

    pltpu.sync_copy(counts_h.at[pl.ds(w * 16, 16)], cb)
    pcnt = cb[pl.ds(0, 16)][0]
    nch = pcnt // 256

    def _proc_rmw(_):
        def _cp(t, _):
            gpg[pl.ds(t * 16, 16)] = gpx[pl.ds(t * 16, 16)]
            return 0
        lax.fori_loop(0, GCH // 16, _cp, 0)
        pltpu.async_copy(p1_h.at[gpg], rb, sem).wait()

        def _k(k, _):
            ll = glx[pl.ds(k * 16, 16)]
            llc = ll * C1
            cnts, _lm = plsc.scan_count(ll)
            mc = jnp.max(cnts)

            def _kk(kk, _):
                mk = cnts == kk
                for c in range(C1):
                    cc = jnp.full((16,), c, jnp.int32)
                    old = plsc.load_gather(canvas, [llc + c], mask=mk)
                    val = plsc.load_gather(rb, [k * 16 + lanes, cc])
                    plsc.store_scatter(canvas, [llc + c], jnp.maximum(old, val), mask=mk)
                return 0
            lax.fori_loop(1, mc + 1, _kk, 0)
            return 0
        lax.fori_loop(0, GCH // 16, _k, 0)
        return 0

    def _proc_d(_):
        def _cp(t, _):
            gpg[pl.ds(t * 16, 16)] = gpx[pl.ds(t * 16, 16)]
            return 0
        lax.fori_loop(0, GCH // 16, _cp, 0)
        pltpu.async_copy(p1_h.at[gpg], vb2, sem).wait()

        def _k(k, _):
            ll = glx[pl.ds(k * 16, 16)]
            llc = ll * C1
            pos = gpx[pl.ds(k * 16, 16)]
            si[pl.ds(k * 16, 16)] = jnp.where(ll == DUMP1, N, pos)
            for c in range(C1):
                cc = jnp.full((16,), c, jnp.int32)
                v = plsc.load_gather(canvas, [llc + c])
                plsc.store_scatter(vb2, [k * 16 + lanes, cc + C1], v)
            return 0
        lax.fori_loop(0, GCH // 16, _k, 0)
        pltpu.async_copy(vb2, x2_h.at[si], sem).wait()
        return 0

    def _sweep(s, proc):
        sbase = s * SUB1

        def _consume(j, gc):
            pltpu.sync_copy(lists_h.at[pl.ds(w * CAP + j * 256, 256)], lch)

            def _v(i, gc):
                pk = lch[pl.ds(i * 16, 16)]
                ll = lax.shift_right_logical(pk, 18) - sbase
                pos = jnp.bitwise_and(pk, 0x3FFFF)
                m = jnp.logical_and(ll >= 0, ll < SUB1)
                offs = plsc.cumsum(jnp.where(m, 1, 0))
                slots = gc + offs - 1
                plsc.store_scatter(gpx, [slots], pos, mask=m)
                plsc.store_scatter(glx, [slots], ll, mask=m)
                gc = gc + jnp.max(offs)

                def _flush(g):
                    proc(0)
                    gpx[pl.ds(0, 16)] = gpx[pl.ds(GCH, 16)]
                    glx[pl.ds(0, 16)] = glx[pl.ds(GCH, 16)]
                    return g - GCH

                return lax.cond(gc >= GCH, _flush, lambda g: g, gc)
            return lax.fori_loop(0, 16, _v, gc)

        def _ip(i, _):
            gpx[pl.ds(i * 16, 16)] = jnp.zeros((16,), jnp.int32)
            glx[pl.ds(i * 16, 16)] = jnp.full((16,), DUMP1, jnp.int32)
            return 0
        lax.fori_loop(0, (GCH + 16) // 16, _ip, 0)
        gc = lax.fori_loop(0, nch, _consume, 0)
        lax.cond(gc > 0, proc, lambda _: 0, 0)
        return 0

    def _sub(s, _):
        def _zc(i, _):
            canvas[pl.ds(i * 16, 16)] = jnp.zeros((16,), jnp.float32)
            return 0
        lax.fori_loop(0, CV1 * C1 // 16, _zc, 0)
        _sweep(s, _proc_rmw)
        _sweep(s, _proc_d)
        return 0
    lax.fori_loop(0, NS1, _sub, 0)


@functools.partial(
    pl.kernel,
    out_type=jax.ShapeDtypeStruct((L * C2,), jnp.float32),
    mesh=_mesh,
    compiler_params=_scp,
    scratch_types=[
        pltpu.VMEM((256,), jnp.int32),
        pltpu.VMEM((GCH + 16,), jnp.int32),
        pltpu.VMEM((GCH + 16,), jnp.int32),
        pltpu.VMEM((GCH,), jnp.int32),
        pltpu.VMEM((GCH, C2), jnp.float32),
        pltpu.VMEM((CV2 * C2,), jnp.float32),
        pltpu.VMEM((16,), jnp.int32),
        pltpu.SemaphoreType.DMA,
    ],
)
def _sc_c(p2_h, lists_h, counts_h, out_h,
          lch, gpx, glx, gpg, rb, canvas, cb, sem):
    cid = lax.axis_index("c")
    sid = lax.axis_index("s")
    w = cid * 16 + sid
    lanes = lax.iota(jnp.int32, 16)

    pltpu.sync_copy(counts_h.at[pl.ds(w * 16, 16)], cb)
    pcnt = cb[pl.ds(0, 16)][0]
    nch = pcnt // 256

    def _proc_rmw(_):
        def _cp(t, _):
            gpg[pl.ds(t * 16, 16)] = gpx[pl.ds(t * 16, 16)]
            return 0
        lax.fori_loop(0, GCH // 16, _cp, 0)
        pltpu.async_copy(p2_h.at[gpg], rb, sem).wait()

        def _k(k, _):
            ll = glx[pl.ds(k * 16, 16)]
            llc = ll * C2
            cnts, _lm = plsc.scan_count(ll)
            mc = jnp.max(cnts)

            def _kk(kk, _):
                mk = cnts == kk
                for c in range(C2):
                    cc = jnp.full((16,), c, jnp.int32)
                    old = plsc.load_gather(canvas, [llc + c], mask=mk)
                    val = plsc.load_gather(rb, [k * 16 + lanes, cc])
                    plsc.store_scatter(canvas, [llc + c], jnp.maximum(old, val), mask=mk)
                return 0
            lax.fori_loop(1, mc + 1, _kk, 0)
            return 0
        lax.fori_loop(0, GCH // 16, _k, 0)
        return 0

    def _sub(s, _):
        sbase = s * SUB2

        def _zc(i, _):
            canvas[pl.ds(i * 16, 16)] = jnp.zeros((16,), jnp.float32)
            return 0
        lax.fori_loop(0, CV2 * C2 // 16, _zc, 0)

        def _ip(i, _):
            gpx[pl.ds(i * 16, 16)] = jnp.zeros((16,), jnp.int32)
            glx[pl.ds(i * 16, 16)] = jnp.full((16,), DUMP2, jnp.int32)
            return 0
        lax.fori_loop(0, (GCH + 16) // 16, _ip, 0)

        def _consume(j, gc):
            pltpu.sync_copy(lists_h.at[pl.ds(w * CAP + j * 256, 256)], lch)

            def _v(i, gc):
                pk = lch[pl.ds(i * 16, 16)]
                ll = lax.shift_right_logical(pk, 18) - sbase
                pos = jnp.bitwise_and(pk, 0x3FFFF)
                m = jnp.logical_and(ll >= 0, ll < SUB2)
                offs = plsc.cumsum(jnp.where(m, 1, 0))
                slots = gc + offs - 1
                plsc.store_scatter(gpx, [slots], pos, mask=m)
                plsc.store_scatter(glx, [slots], ll, mask=m)
                gc = gc + jnp.max(offs)

                def _flush(g):
                    _proc_rmw(0)
                    gpx[pl.ds(0, 16)] = gpx[pl.ds(GCH, 16)]
                    glx[pl.ds(0, 16)] = glx[pl.ds(GCH, 16)]
                    return g - GCH

                return lax.cond(gc >= GCH, _flush, lambda g: g, gc)
            return lax.fori_loop(0, 16, _v, gc)

        gc = lax.fori_loop(0, nch, _consume, 0)
        lax.cond(gc > 0, _proc_rmw, lambda _: 0, 0)
        pltpu.sync_copy(canvas.at[pl.ds(0, SUB2 * C2)],
                        out_h.at[pl.ds((w * VR + s * SUB2) * C2, SUB2 * C2)])
        return 0
    lax.fori_loop(0, NS2, _sub, 0)


def _bn_coeffs(G, W, g, b, nfeat):
    n = jnp.float32(N)
    S = G[:nfeat, :nfeat] / n
    mu = G[nfeat, :nfeat] / n
    m = mu @ W
    E2 = jnp.sum(W * (S @ W), axis=0)
    var = E2 - m * m
    rstd = 1.0 / jnp.sqrt(var + 1e-3)
    scale = g * rstd
    shift = b - m * scale
    return scale[None, :], shift[None, :]


def kernel(points, W1, g1, b1, W2, g2, b2, batch_size):
    flat2, pt4 = pl.pallas_call(
        _prep_body,
        grid=(NB,),
        in_specs=[pl.BlockSpec((BN, 5), lambda i: (i, 0))],
        out_specs=[pl.BlockSpec((BN, 1), lambda i: (i, 0)),
                   pl.BlockSpec((BN, 4), lambda i: (i, 0))],
        out_shape=[jax.ShapeDtypeStruct((N, 1), jnp.int32),
                   jax.ShapeDtypeStruct((N, 4), jnp.float32)],
    )(points)
    flat = flat2.reshape(N)
    pt4f = pt4.reshape(N * 4)

    fclu, _vcan, lists, counts = _sc_a(flat, pt4f)
    fclu2 = fclu.reshape(N, 4)

    G1 = pl.pallas_call(
        _gram1_body,
        grid=(NB,),
        in_specs=[pl.BlockSpec((BN, 5), lambda i: (i, 0)),
                  pl.BlockSpec((BN, 4), lambda i: (i, 0))],
        out_specs=pl.BlockSpec((16, 16), lambda i: (0, 0)),
        out_shape=jax.ShapeDtypeStruct((16, 16), jnp.float32),
        scratch_shapes=[pltpu.VMEM((16, 16), jnp.float32)],
    )(points, fclu2)
    sc1, sh1 = _bn_coeffs(G1, W1, g1, b1, 10)

    p1p = pl.pallas_call(
        _apply1_body,
        grid=(NB,),
        in_specs=[pl.BlockSpec((BN, 5), lambda i: (i, 0)),
                  pl.BlockSpec((BN, 4), lambda i: (i, 0)),
                  pl.BlockSpec((10, C1), lambda i: (0, 0)),
                  pl.BlockSpec((1, C1), lambda i: (0, 0)),
                  pl.BlockSpec((1, C1), lambda i: (0, 0))],
        out_specs=pl.BlockSpec((BN, C2), lambda i: (i, 0)),
        out_shape=jax.ShapeDtypeStruct((N, C2), jnp.float32),
    )(points, fclu2, W1, sc1, sh1)

    X2 = _sc_b(p1p, lists, counts)

    G2 = pl.pallas_call(
        _gram2_body,
        grid=(NB,),
        in_specs=[pl.BlockSpec((BN, C2), lambda i: (i, 0))],
        out_specs=pl.BlockSpec((136, 136), lambda i: (0, 0)),
        out_shape=jax.ShapeDtypeStruct((136, 136), jnp.float32),
        scratch_shapes=[pltpu.VMEM((136, 136), jnp.float32)],
    )(X2)
    sc2, sh2 = _bn_coeffs(G2, W2, g2, b2, C2)

    p2 = pl.pallas_call(
        _apply2_body,
        grid=(NB,),
        in_specs=[pl.BlockSpec((BN, C2), lambda i: (i, 0)),
                  pl.BlockSpec((C2, C2), lambda i: (0, 0)),
                  pl.BlockSpec((1, C2), lambda i: (0, 0)),
                  pl.BlockSpec((1, C2), lambda i: (0, 0))],
        out_specs=pl.BlockSpec((BN, C2), lambda i: (i, 0)),
        out_shape=jax.ShapeDtypeStruct((N, C2), jnp.float32),
    )(X2, W2, sc2, sh2)

    return _sc_c(p2, lists, counts).reshape(L, C2)


# speedup vs baseline: 1.9960x; 1.8701x over previous
import functools
import jax
import jax.numpy as jnp
from jax import lax
from jax.experimental import pallas as pl
from jax.experimental.pallas import tpu as pltpu, tpu_sc as plsc

VX, VY, VZ = 0.2, 0.2, 4.0
PCR = (0.0, -40.0, -3.0, 70.4, 40.0, 1.0)
CZ, CY, CX = 1, 400, 352
B = 2
N = 262144
C1, C2 = 64, 128
X_OFF = VX / 2 + PCR[0]
Y_OFF = VY / 2 + PCR[1]
Z_OFF = VZ / 2 + PCR[2]
L = B * CZ * CY * CX
HL = L // 2
NW = 32
NHALF = N // 2
WPTS = N // NW
CH = 4096
VR = HL // 16
F = 2048
CAP = 133120
SUB1, NS1, CV1, DUMP1 = 1100, 8, 1152, 1100
SUB2, NS2, CV2, DUMP2 = 550, 16, 576, 550
GCH = 128
BN = 2048
NB = N // BN

_mesh = plsc.VectorSubcoreMesh(core_axis_name="c", subcore_axis_name="s")
_scp = pltpu.CompilerParams(needs_layout_passes=False)


def _prep_body(p_ref, flat_ref, pt4_ref):
    x = p_ref[:, 1]
    y = p_ref[:, 2]
    z = p_ref[:, 3]
    b = p_ref[:, 0].astype(jnp.int32)
    cx = jnp.clip(jnp.floor((x - PCR[0]) / VX).astype(jnp.int32), 0, CX - 1)
    cy = jnp.clip(jnp.floor((y - PCR[1]) / VY).astype(jnp.int32), 0, CY - 1)
    cz = jnp.clip(jnp.floor((z - PCR[2]) / VZ).astype(jnp.int32), 0, CZ - 1)
    flat = ((b * CZ + cz) * CY + cy) * CX + cx
    flat_ref[...] = flat[:, None]
    pt4_ref[...] = jnp.stack([x, y, z, jnp.ones_like(x)], axis=1)


def _x1(p_ref, fc_ref):
    x = p_ref[:, 1]
    y = p_ref[:, 2]
    z = p_ref[:, 3]
    cx = jnp.clip(jnp.floor((x - PCR[0]) / VX).astype(jnp.int32), 0, CX - 1)
    cy = jnp.clip(jnp.floor((y - PCR[1]) / VY).astype(jnp.int32), 0, CY - 1)
    cz = jnp.clip(jnp.floor((z - PCR[2]) / VZ).astype(jnp.int32), 0, CZ - 1)
    fcen = jnp.stack([
        x - (cx.astype(jnp.float32) * VX + X_OFF),
        y - (cy.astype(jnp.float32) * VY + Y_OFF),
        z - (cz.astype(jnp.float32) * VZ + Z_OFF),
    ], axis=1)
    return jnp.concatenate([p_ref[:, 1:5], fc_ref[:, 0:3], fcen], axis=1)


def _gram1_body(p_ref, fc_ref, g_ref, acc):
    i = pl.program_id(0)

    @pl.when(i == 0)
    def _():
        acc[...] = jnp.zeros_like(acc)

    X = _x1(p_ref, fc_ref)
    ones = jnp.ones((X.shape[0], 1), jnp.float32)
    zeros = jnp.zeros((X.shape[0], 5), jnp.float32)
    Xe = jnp.concatenate([X, ones, zeros], axis=1)
    acc[...] += lax.dot_general(Xe, Xe, (((0,), (0,)), ((), ())),
                                preferred_element_type=jnp.float32)

    @pl.when(i == NB - 1)
    def _():
        g_ref[...] = acc[...]


def _apply1_body(p_ref, fc_ref, w_ref, sc_ref, sh_ref, o_ref):
    X = _x1(p_ref, fc_ref)
    h = jnp.dot(X, w_ref[...], preferred_element_type=jnp.float32)
    p1 = jax.nn.relu(h * sc_ref[...] + sh_ref[...])
    o_ref[...] = jnp.concatenate(
        [p1, jnp.zeros((p1.shape[0], C2 - C1), jnp.float32)], axis=1)


def _gram2_body(x2_ref, g_ref, acc):
    i = pl.program_id(0)

    @pl.when(i == 0)
    def _():
        acc[...] = jnp.zeros_like(acc)

    X = x2_ref[...]
    ones = jnp.ones((X.shape[0], 1), jnp.float32)
    zeros = jnp.zeros((X.shape[0], 7), jnp.float32)
    Xe = jnp.concatenate([X, ones, zeros], axis=1)
    acc[...] += lax.dot_general(Xe, Xe, (((0,), (0,)), ((), ())),
                                preferred_element_type=jnp.float32)

    @pl.when(i == NB - 1)
    def _():
        g_ref[...] = acc[...]


def _apply2_body(x2_ref, w_ref, sc_ref, sh_ref, o_ref):
    h = jnp.dot(x2_ref[...], w_ref[...], preferred_element_type=jnp.float32)
    o_ref[...] = jax.nn.relu(h * sc_ref[...] + sh_ref[...])


@functools.partial(
    pl.kernel,
    out_type=(
        jax.ShapeDtypeStruct((N * 4,), jnp.float32),
        jax.ShapeDtypeStruct((L * 4,), jnp.float32),
        jax.ShapeDtypeStruct((NW * CAP,), jnp.int32),
        jax.ShapeDtypeStruct((NW * 16,), jnp.int32),
    ),
    mesh=_mesh,
    compiler_params=_scp,
    scratch_types=[
        pltpu.VMEM((CH,), jnp.int32),
        pltpu.VMEM((CH * 4,), jnp.float32),
        pltpu.VMEM((CH * 4,), jnp.int32),
        pltpu.VMEM((CH * 4,), jnp.float32),
        pltpu.VMEM((CH * 4,), jnp.float32),
        pltpu.VMEM((F + 16,), jnp.int32),
        pltpu.VMEM((16,), jnp.int32),
        pltpu.VMEM((VR // 2,), jnp.float32),
        pltpu.VMEM_SHARED((HL * 4,), jnp.float32),
        pltpu.SemaphoreType.DMA,
    ],
)
def _sc_a(flat_h, pt4_h, fclu_h, vcan_h, lists_h, counts_h,
          fiv, valw, eix, rows, ob, lbuf, cb, zb, canvas, sem):
    cid = lax.axis_index("c")
    sid = lax.axis_index("s")
    w = cid * 16 + sid
    lanes = lax.iota(jnp.int32, 16)

    def _zb(i, _):
        zb[pl.ds(i * 16, 16)] = jnp.zeros((16,), jnp.float32)
        return 0
    lax.fori_loop(0, (VR // 2) // 16, _zb, 0)

    def _zc(i, _):
        pltpu.sync_copy(zb, canvas.at[pl.ds(sid * VR * 4 + i * (VR // 2), VR // 2)])
        return 0
    lax.fori_loop(0, 8, _zc, 0)
    plsc.subcore_barrier()

    for t in range(WPTS // CH):
        base = w * WPTS + t * CH
        pltpu.sync_copy(flat_h.at[pl.ds(base, CH)], fiv)
        pltpu.sync_copy(pt4_h.at[pl.ds(base * 4, CH * 4)], valw)

        def _bx(i, _):
            f = fiv[pl.ds(i * 16, 16)]
            e = (f - cid * HL) * 4
            for c in range(4):
                plsc.store_scatter(eix, [lanes * 4 + (i * 64 + c)], e + c)
            return 0
        lax.fori_loop(0, CH // 16, _bx, 0)
        pltpu.sync_copy(valw, canvas.at[eix], add=True)
    plsc.subcore_barrier()

    def _rc(i, _):
        pltpu.sync_copy(canvas.at[pl.ds(sid * VR * 4 + i * (VR // 2), VR // 2)], zb)
        pltpu.sync_copy(zb, vcan_h.at[pl.ds(cid * HL * 4 + sid * VR * 4 + i * (VR // 2), VR // 2)])
        return 0
    lax.fori_loop(0, 8, _rc, 0)

    for t in range(WPTS // CH):
        base = w * WPTS + t * CH
        pltpu.sync_copy(flat_h.at[pl.ds(base, CH)], fiv)
        pltpu.sync_copy(pt4_h.at[pl.ds(base * 4, CH * 4)], valw)

        def _bg(i, _):
            f = fiv[pl.ds(i * 16, 16)]
            e = f * 4
            for c in range(4):
                plsc.store_scatter(eix, [lanes * 4 + (i * 64 + c)], e + c)
            return 0
        lax.fori_loop(0, CH // 16, _bg, 0)
        pltpu.async_copy(vcan_h.at[eix], rows, sem).wait()

        def _fc(i, _):
            i64 = i * 64
            sx = plsc.load_gather(rows, [i64 + lanes * 4 + 0])
            sy = plsc.load_gather(rows, [i64 + lanes * 4 + 1])
            sz = plsc.load_gather(rows, [i64 + lanes * 4 + 2])
            cv = plsc.load_gather(rows, [i64 + lanes * 4 + 3])
            x = plsc.load_gather(valw, [i64 + lanes * 4 + 0])
            y = plsc.load_gather(valw, [i64 + lanes * 4 + 1])
            z = plsc.load_gather(valw, [i64 + lanes * 4 + 2])
            inv = 1.0 / jnp.maximum(cv, 1.0)
            plsc.store_scatter(ob, [i64 + lanes * 4 + 0], x - sx * inv)
            plsc.store_scatter(ob, [i64 + lanes * 4 + 1], y - sy * inv)
            plsc.store_scatter(ob, [i64 + lanes * 4 + 2], z - sz * inv)
            plsc.store_scatter(ob, [i64 + lanes * 4 + 3], jnp.zeros((16,), jnp.float32))
            return 0
        lax.fori_loop(0, CH // 16, _fc, 0)
        pltpu.sync_copy(ob, fclu_h.at[pl.ds(base * 4, CH * 4)])

    dump_pat = jnp.full((16,), VR, jnp.int32) << 18

    def _lb(i, _):
        lbuf[pl.ds(i * 16, 16)] = dump_pat
        return 0
    lax.fori_loop(0, (F + 16) // 16, _lb, 0)

    bb = cid * NHALF

    def _chunk(t, carry):
        pltpu.sync_copy(flat_h.at[pl.ds(bb + t * CH, CH)], fiv)

        def _v(i, cr):
            cnt, fl = cr
            f = fiv[pl.ds(i * 16, 16)]
            lid = f - w * VR
            m = jnp.logical_and(lid >= 0, lid < VR)
            pos = bb + t * CH + i * 16 + lanes
            packed = jnp.bitwise_or(pos, lid << 18)
            offs = plsc.cumsum(jnp.where(m, 1, 0))
            slots = cnt + offs - 1
            plsc.store_scatter(lbuf, [slots], packed, mask=m)
            cnt = cnt + jnp.max(offs)

            def _flush(c):
                cc, ff = c
                ffa = pl.multiple_of(ff, F)
                pltpu.sync_copy(lbuf.at[pl.ds(0, F)], lists_h.at[pl.ds(w * CAP + ffa, F)])
                tail = lbuf[pl.ds(F, 16)]
                lbuf[pl.ds(0, 16)] = tail
                return (cc - F, ff + F)

            return lax.cond(cnt >= F, _flush, lambda c: c, (cnt, fl))
        return lax.fori_loop(0, CH // 16, _v, carry)

    cnt, fl = lax.fori_loop(0, NHALF // CH, _chunk, (0, 0))
    fla = pl.multiple_of(fl, F)
    pltpu.sync_copy(lbuf.at[pl.ds(0, F)], lists_h.at[pl.ds(w * CAP + fla, F)])
    padded = fl + ((cnt + 255) // 256) * 256
    cb[pl.ds(0, 16)] = jnp.broadcast_to(padded, (16,)).astype(jnp.int32)
    pltpu.sync_copy(cb, counts_h.at[pl.ds(w * 16, 16)])


@functools.partial(
    pl.kernel,
    out_type=jax.ShapeDtypeStruct((N + 8, C2), jnp.float32),
    mesh=_mesh,
    compiler_params=_scp,
    scratch_types=[
        pltpu.VMEM((256,), jnp.int32),
        pltpu.VMEM((GCH + 16,), jnp.int32),
        pltpu.VMEM((GCH + 16,), jnp.int32),
        pltpu.VMEM((GCH,), jnp.int32),
        pltpu.VMEM((GCH,), jnp.int32),
        pltpu.VMEM((GCH, C2), jnp.float32),
        pltpu.VMEM((GCH, C2), jnp.float32),
        pltpu.VMEM((CV1 * C1,), jnp.float32),
        pltpu.VMEM((16,), jnp.int32),
        pltpu.SemaphoreType.DMA,
    ],
)
def _sc_b(p1_h, lists_h, counts_h, x2_h,
          lch, gpx, glx, gpg, si, rb, vb2, canvas, cb, sem):
    cid = lax.axis_index("c")
    sid = lax.axis_index("s")
    w = cid * 16 + sid
    lanes = lax.iota(jnp.int32, 16)

    pltpu.sync_copy(counts_h.at[pl.ds(w * 16, 16)], cb)
    pcnt = cb[pl.ds(0, 16)][0]
    nch = pcnt // 256

    def _proc_rmw(_):
        def _cp(t, _):
            gpg[pl.ds(t * 16, 16)] = gpx[pl.ds(t * 16, 16)]
            return 0
        lax.fori_loop(0, GCH // 16, _cp, 0)
        pltpu.async_copy(p1_h.at[gpg], rb, sem).wait()

        def _k(k, _):
            llv = glx[pl.ds(k * 16, 16)] * C1
            for j in range(16):
                base = pl.multiple_of(llv[j], C1)
                for c in range(C1 // 16):
                    old = canvas[pl.ds(base + c * 16, 16)]
                    val = rb[k * 16 + j, pl.ds(c * 16, 16)]
                    canvas[pl.ds(base + c * 16, 16)] = jnp.maximum(old, val)
            return 0
        lax.fori_loop(0, GCH // 16, _k, 0)
        return 0

    def _proc_d(_):
        def _cp(t, _):
            gpg[pl.ds(t * 16, 16)] = gpx[pl.ds(t * 16, 16)]
            return 0
        lax.fori_loop(0, GCH // 16, _cp, 0)
        pltpu.async_copy(p1_h.at[gpg], vb2, sem).wait()

        def _k(k, _):
            ll = glx[pl.ds(k * 16, 16)]
            llv = ll * C1
            pos = gpx[pl.ds(k * 16, 16)]
            si[pl.ds(k * 16, 16)] = jnp.where(ll == DUMP1, N, pos)
            for j in range(16):
                base = pl.multiple_of(llv[j], C1)
                for c in range(C1 // 16):
                    vb2[k * 16 + j, pl.ds(C1 + c * 16, 16)] = canvas[pl.ds(base + c * 16, 16)]
            return 0
        lax.fori_loop(0, GCH // 16, _k, 0)
        pltpu.async_copy(vb2, x2_h.at[si], sem).wait()
        return 0

    def _sweep(s, proc):
        sbase = s * SUB1

        def _consume(j, gc):
            pltpu.sync_copy(lists_h.at[pl.ds(w * CAP + j * 256, 256)], lch)

            def _v(i, gc):
                pk = lch[pl.ds(i * 16, 16)]
                ll = lax.shift_right_logical(pk, 18) - sbase
                pos = jnp.bitwise_and(pk, 0x3FFFF)
                m = jnp.logical_and(ll >= 0, ll < SUB1)
                offs = plsc.cumsum(jnp.where(m, 1, 0))
                slots = gc + offs - 1
                plsc.store_scatter(gpx, [slots], pos, mask=m)
                plsc.store_scatter(glx, [slots], ll, mask=m)
                gc = gc + jnp.max(offs)

                def _flush(g):
                    proc(0)
                    gpx[pl.ds(0, 16)] = gpx[pl.ds(GCH, 16)]
                    glx[pl.ds(0, 16)] = glx[pl.ds(GCH, 16)]
                    return g - GCH

                return lax.cond(gc >= GCH, _flush, lambda g: g, gc)
            return lax.fori_loop(0, 16, _v, gc)

        def _ip(i, _):
            gpx[pl.ds(i * 16, 16)] = jnp.zeros((16,), jnp.int32)
            glx[pl.ds(i * 16, 16)] = jnp.full((16,), DUMP1, jnp.int32)
            return 0
        lax.fori_loop(0, (GCH + 16) // 16, _ip, 0)
        gc = lax.fori_loop(0, nch, _consume, 0)
        lax.cond(gc > 0, proc, lambda _: 0, 0)
        return 0

    def _sub(s, _):
        def _zc(i, _):
            canvas[pl.ds(i * 16, 16)] = jnp.zeros((16,), jnp.float32)
            return 0
        lax.fori_loop(0, CV1 * C1 // 16, _zc, 0)
        _sweep(s, _proc_rmw)
        _sweep(s, _proc_d)
        return 0
    lax.fori_loop(0, NS1, _sub, 0)


@functools.partial(
    pl.kernel,
    out_type=jax.ShapeDtypeStruct((L * C2,), jnp.float32),
    mesh=_mesh,
    compiler_params=_scp,
    scratch_types=[
        pltpu.VMEM((256,), jnp.int32),
        pltpu.VMEM((GCH + 16,), jnp.int32),
        pltpu.VMEM((GCH + 16,), jnp.int32),
        pltpu.VMEM((GCH,), jnp.int32),
        pltpu.VMEM((GCH, C2), jnp.float32),
        pltpu.VMEM((CV2 * C2,), jnp.float32),
        pltpu.VMEM((16,), jnp.int32),
        pltpu.SemaphoreType.DMA,
    ],
)
def _sc_c(p2_h, lists_h, counts_h, out_h,
          lch, gpx, glx, gpg, rb, canvas, cb, sem):
    cid = lax.axis_index("c")
    sid = lax.axis_index("s")
    w = cid * 16 + sid
    lanes = lax.iota(jnp.int32, 16)

    pltpu.sync_copy(counts_h.at[pl.ds(w * 16, 16)], cb)
    pcnt = cb[pl.ds(0, 16)][0]
    nch = pcnt // 256

    def _proc_rmw(_):
        def _cp(t, _):
            gpg[pl.ds(t * 16, 16)] = gpx[pl.ds(t * 16, 16)]
            return 0
        lax.fori_loop(0, GCH // 16, _cp, 0)
        pltpu.async_copy(p2_h.at[gpg], rb, sem).wait()

        def _k(k, _):
            llv = glx[pl.ds(k * 16, 16)] * C2
            for j in range(16):
                base = pl.multiple_of(llv[j], C2)
                for c in range(C2 // 16):
                    old = canvas[pl.ds(base + c * 16, 16)]
                    val = rb[k * 16 + j, pl.ds(c * 16, 16)]
                    canvas[pl.ds(base + c * 16, 16)] = jnp.maximum(old, val)
            return 0
        lax.fori_loop(0, GCH // 16, _k, 0)
        return 0

    def _sub(s, _):
        sbase = s * SUB2

        def _zc(i, _):
            canvas[pl.ds(i * 16, 16)] = jnp.zeros((16,), jnp.float32)
            return 0
        lax.fori_loop(0, CV2 * C2 // 16, _zc, 0)

        def _ip(i, _):
            gpx[pl.ds(i * 16, 16)] = jnp.zeros((16,), jnp.int32)
            glx[pl.ds(i * 16, 16)] = jnp.full((16,), DUMP2, jnp.int32)
            return 0
        lax.fori_loop(0, (GCH + 16) // 16, _ip, 0)

        def _consume(j, gc):
            pltpu.sync_copy(lists_h.at[pl.ds(w * CAP + j * 256, 256)], lch)

            def _v(i, gc):
                pk = lch[pl.ds(i * 16, 16)]
                ll = lax.shift_right_logical(pk, 18) - sbase
                pos = jnp.bitwise_and(pk, 0x3FFFF)
                m = jnp.logical_and(ll >= 0, ll < SUB2)
                offs = plsc.cumsum(jnp.where(m, 1, 0))
                slots = gc + offs - 1
                plsc.store_scatter(gpx, [slots], pos, mask=m)
                plsc.store_scatter(glx, [slots], ll, mask=m)
                gc = gc + jnp.max(offs)

                def _flush(g):
                    _proc_rmw(0)
                    gpx[pl.ds(0, 16)] = gpx[pl.ds(GCH, 16)]
                    glx[pl.ds(0, 16)] = glx[pl.ds(GCH, 16)]
                    return g - GCH

                return lax.cond(gc >= GCH, _flush, lambda g: g, gc)
            return lax.fori_loop(0, 16, _v, gc)

        gc = lax.fori_loop(0, nch, _consume, 0)
        lax.cond(gc > 0, _proc_rmw, lambda _: 0, 0)
        pltpu.sync_copy(canvas.at[pl.ds(0, SUB2 * C2)],
                        out_h.at[pl.ds((w * VR + s * SUB2) * C2, SUB2 * C2)])
        return 0
    lax.fori_loop(0, NS2, _sub, 0)


def _bn_coeffs(G, W, g, b, nfeat):
    n = jnp.float32(N)
    S = G[:nfeat, :nfeat] / n
    mu = G[nfeat, :nfeat] / n
    m = mu @ W
    E2 = jnp.sum(W * (S @ W), axis=0)
    var = E2 - m * m
    rstd = 1.0 / jnp.sqrt(var + 1e-3)
    scale = g * rstd
    shift = b - m * scale
    return scale[None, :], shift[None, :]


def kernel(points, W1, g1, b1, W2, g2, b2, batch_size):
    flat2, pt4 = pl.pallas_call(
        _prep_body,
        grid=(NB,),
        in_specs=[pl.BlockSpec((BN, 5), lambda i: (i, 0))],
        out_specs=[pl.BlockSpec((BN, 1), lambda i: (i, 0)),
                   pl.BlockSpec((BN, 4), lambda i: (i, 0))],
        out_shape=[jax.ShapeDtypeStruct((N, 1), jnp.int32),
                   jax.ShapeDtypeStruct((N, 4), jnp.float32)],
    )(points)
    flat = flat2.reshape(N)
    pt4f = pt4.reshape(N * 4)

    fclu, _vcan, lists, counts = _sc_a(flat, pt4f)
    fclu2 = fclu.reshape(N, 4)

    G1 = pl.pallas_call(
        _gram1_body,
        grid=(NB,),
        in_specs=[pl.BlockSpec((BN, 5), lambda i: (i, 0)),
                  pl.BlockSpec((BN, 4), lambda i: (i, 0))],
        out_specs=pl.BlockSpec((16, 16), lambda i: (0, 0)),
        out_shape=jax.ShapeDtypeStruct((16, 16), jnp.float32),
        scratch_shapes=[pltpu.VMEM((16, 16), jnp.float32)],
    )(points, fclu2)
    sc1, sh1 = _bn_coeffs(G1, W1, g1, b1, 10)

    p1p = pl.pallas_call(
        _apply1_body,
        grid=(NB,),
        in_specs=[pl.BlockSpec((BN, 5), lambda i: (i, 0)),
                  pl.BlockSpec((BN, 4), lambda i: (i, 0)),
                  pl.BlockSpec((10, C1), lambda i: (0, 0)),
                  pl.BlockSpec((1, C1), lambda i: (0, 0)),
                  pl.BlockSpec((1, C1), lambda i: (0, 0))],
        out_specs=pl.BlockSpec((BN, C2), lambda i: (i, 0)),
        out_shape=jax.ShapeDtypeStruct((N, C2), jnp.float32),
    )(points, fclu2, W1, sc1, sh1)

    X2 = _sc_b(p1p, lists, counts)

    G2 = pl.pallas_call(
        _gram2_body,
        grid=(NB,),
        in_specs=[pl.BlockSpec((BN, C2), lambda i: (i, 0))],
        out_specs=pl.BlockSpec((136, 136), lambda i: (0, 0)),
        out_shape=jax.ShapeDtypeStruct((136, 136), jnp.float32),
        scratch_shapes=[pltpu.VMEM((136, 136), jnp.float32)],
    )(X2)
    sc2, sh2 = _bn_coeffs(G2, W2, g2, b2, C2)

    p2 = pl.pallas_call(
        _apply2_body,
        grid=(NB,),
        in_specs=[pl.BlockSpec((BN, C2), lambda i: (i, 0)),
                  pl.BlockSpec((C2, C2), lambda i: (0, 0)),
                  pl.BlockSpec((1, C2), lambda i: (0, 0)),
                  pl.BlockSpec((1, C2), lambda i: (0, 0))],
        out_specs=pl.BlockSpec((BN, C2), lambda i: (i, 0)),
        out_shape=jax.ShapeDtypeStruct((N, C2), jnp.float32),
    )(X2, W2, sc2, sh2)

    return _sc_c(p2, lists, counts).reshape(L, C2)
